# Initial kernel scaffold; baseline (speedup 1.0000x reference)
import jax
import jax.numpy as jnp
from jax.experimental import pallas as pl


def kernel(x, W, expert_bias):
    raise NotImplementedError("write your pallas kernel here")


# speedup vs baseline: 1.2959x; 1.2959x over previous
import functools

import jax
import jax.numpy as jnp
from jax.experimental import pallas as pl
from jax.experimental.pallas import tpu as pltpu

NUM_EXPERTS = 64
TOP_K = 8
D_MODEL = 4096
TOKENS = 16384

T_BLK = 1024


def _router_block(x_ref, wt_ref, b_ref, w_out, i_out, psum_out, zsum_out):
    logits = jnp.dot(
        x_ref[...], wt_ref[...], preferred_element_type=jnp.float32
    ) + b_ref[...]

    zsum_out[0, 0, :] = jnp.sum(logits * logits, axis=0)

    m = jnp.max(logits, axis=-1, keepdims=True)
    e = jnp.exp(logits - m)
    s = jnp.sum(e, axis=-1, keepdims=True)
    probs = e / s

    psum_out[0, 0, :] = jnp.sum(probs, axis=0)

    lane = jax.lax.broadcasted_iota(jnp.int32, probs.shape, 1)
    vals = probs
    ws = []
    idxs = []
    for _ in range(TOP_K):
        mk = jnp.max(vals, axis=-1, keepdims=True)
        is_mk = vals >= mk
        idx = jnp.min(
            jnp.where(is_mk, lane, NUM_EXPERTS), axis=-1, keepdims=True
        )
        ws.append(mk)
        idxs.append(idx)
        vals = jnp.where(lane == idx, -1.0, vals)

    w_cat = jnp.concatenate(ws, axis=-1)
    wsum = jnp.sum(w_cat, axis=-1, keepdims=True)
    w_out[...] = w_cat / (wsum + 1e-8)
    i_out[...] = jnp.concatenate(idxs, axis=-1)


@jax.jit
def kernel(x, W, expert_bias):
    grid = TOKENS // T_BLK
    w_t = W.T
    bias = expert_bias.reshape(1, NUM_EXPERTS)

    w_out, i_out, psum, zsum = pl.pallas_call(
        _router_block,
        grid=(grid,),
        in_specs=[
            pl.BlockSpec((T_BLK, D_MODEL), lambda i: (i, 0)),
            pl.BlockSpec((D_MODEL, NUM_EXPERTS), lambda i: (0, 0)),
            pl.BlockSpec((1, NUM_EXPERTS), lambda i: (0, 0)),
        ],
        out_specs=[
            pl.BlockSpec((T_BLK, TOP_K), lambda i: (i, 0)),
            pl.BlockSpec((T_BLK, TOP_K), lambda i: (i, 0)),
            pl.BlockSpec((1, 1, NUM_EXPERTS), lambda i: (i, 0, 0)),
            pl.BlockSpec((1, 1, NUM_EXPERTS), lambda i: (i, 0, 0)),
        ],
        out_shape=[
            jax.ShapeDtypeStruct((TOKENS, TOP_K), jnp.float32),
            jax.ShapeDtypeStruct((TOKENS, TOP_K), jnp.int32),
            jax.ShapeDtypeStruct((grid, 1, NUM_EXPERTS), jnp.float32),
            jax.ShapeDtypeStruct((grid, 1, NUM_EXPERTS), jnp.float32),
        ],
        compiler_params=pltpu.CompilerParams(
            dimension_semantics=("arbitrary",),
        ),
    )(x, w_t, bias)

    tokens_per_expert = jnp.sum(psum, axis=(0, 1)) / TOKENS
    uniform = 1.0 / NUM_EXPERTS
    load_balance_loss = (
        jnp.sum((tokens_per_expert - uniform) ** 2) * NUM_EXPERTS
    )
    router_z_loss = jnp.sum(zsum) / (TOKENS * NUM_EXPERTS) * 0.001
    return (w_out, i_out, load_balance_loss, router_z_loss)


# speedup vs baseline: 1.2984x; 1.0019x over previous
import functools

import jax
import jax.numpy as jnp
from jax.experimental import pallas as pl
from jax.experimental.pallas import tpu as pltpu

NUM_EXPERTS = 64
TOP_K = 8
D_MODEL = 4096
TOKENS = 16384

T_BLK = 1024


def _router_block(x_ref, wt_ref, b_ref, w_out, i_out, psum_out, zsum_out):
    logits = jnp.dot(
        x_ref[...], wt_ref[...], preferred_element_type=jnp.float32
    ) + b_ref[...]

    zsum_out[0, 0, :] = jnp.sum(logits * logits, axis=0)

    m = jnp.max(logits, axis=-1, keepdims=True)
    e = jnp.exp(logits - m)
    s = jnp.sum(e, axis=-1, keepdims=True)
    probs = e / s

    psum_out[0, 0, :] = jnp.sum(probs, axis=0)

    lane = jax.lax.broadcasted_iota(jnp.int32, probs.shape, 1)
    vals = probs
    ws = []
    idxs = []
    for _ in range(TOP_K):
        mk = jnp.max(vals, axis=-1, keepdims=True)
        is_mk = vals >= mk
        idx = jnp.min(
            jnp.where(is_mk, lane, NUM_EXPERTS), axis=-1, keepdims=True
        )
        ws.append(mk)
        idxs.append(idx)
        vals = jnp.where(lane == idx, -1.0, vals)

    w_cat = jnp.concatenate(ws, axis=-1)
    wsum = jnp.sum(w_cat, axis=-1, keepdims=True)
    w_out[...] = w_cat / (wsum + 1e-8)
    i_out[...] = jnp.concatenate(idxs, axis=-1)


@jax.jit
def kernel(x, W, expert_bias):
    grid = TOKENS // T_BLK
    w_t = W.T
    bias = expert_bias.reshape(1, NUM_EXPERTS)

    w_out, i_out, psum, zsum = pl.pallas_call(
        _router_block,
        grid=(grid,),
        in_specs=[
            pl.BlockSpec((T_BLK, D_MODEL), lambda i: (i, 0)),
            pl.BlockSpec((D_MODEL, NUM_EXPERTS), lambda i: (0, 0)),
            pl.BlockSpec((1, NUM_EXPERTS), lambda i: (0, 0)),
        ],
        out_specs=[
            pl.BlockSpec((T_BLK, TOP_K), lambda i: (i, 0)),
            pl.BlockSpec((T_BLK, TOP_K), lambda i: (i, 0)),
            pl.BlockSpec((1, 1, NUM_EXPERTS), lambda i: (i, 0, 0)),
            pl.BlockSpec((1, 1, NUM_EXPERTS), lambda i: (i, 0, 0)),
        ],
        out_shape=[
            jax.ShapeDtypeStruct((TOKENS, TOP_K), jnp.float32),
            jax.ShapeDtypeStruct((TOKENS, TOP_K), jnp.int32),
            jax.ShapeDtypeStruct((grid, 1, NUM_EXPERTS), jnp.float32),
            jax.ShapeDtypeStruct((grid, 1, NUM_EXPERTS), jnp.float32),
        ],
        compiler_params=pltpu.CompilerParams(
            dimension_semantics=("parallel",),
        ),
    )(x, w_t, bias)

    tokens_per_expert = jnp.sum(psum, axis=(0, 1)) / TOKENS
    uniform = 1.0 / NUM_EXPERTS
    load_balance_loss = (
        jnp.sum((tokens_per_expert - uniform) ** 2) * NUM_EXPERTS
    )
    router_z_loss = jnp.sum(zsum) / (TOKENS * NUM_EXPERTS) * 0.001
    return (w_out, i_out, load_balance_loss, router_z_loss)


# speedup vs baseline: 1.4823x; 1.1416x over previous
import jax
import jax.numpy as jnp
from jax.experimental import pallas as pl
from jax.experimental.pallas import tpu as pltpu

NUM_EXPERTS = 64
TOP_K = 8
D_MODEL = 4096
TOKENS = 16384

T_BLK = 1024


def _router_block(x_ref, wt_ref, b_ref, w_out, i_out, psum_out, zsum_out):
    logits = jnp.dot(
        x_ref[...], wt_ref[...], preferred_element_type=jnp.float32
    )

    lt = logits.T + b_ref[...]
    zsum_out[0, 0, :] = jnp.sum(lt * lt, axis=1)

    m = jnp.max(lt, axis=0, keepdims=True)
    e = jnp.exp(lt - m)
    s = jnp.sum(e, axis=0, keepdims=True)
    probs = e / s

    psum_out[0, 0, :] = jnp.sum(probs, axis=1)

    sub = jax.lax.broadcasted_iota(jnp.int32, probs.shape, 0)
    vals = probs
    ws = []
    idxs = []
    for _ in range(TOP_K):
        mk = jnp.max(vals, axis=0, keepdims=True)
        is_mk = vals >= mk
        idx = jnp.min(
            jnp.where(is_mk, sub, NUM_EXPERTS), axis=0, keepdims=True
        )
        ws.append(mk)
        idxs.append(idx)
        vals = jnp.where(sub == idx, -1.0, vals)

    w_cat = jnp.concatenate(ws, axis=0)
    wsum = jnp.sum(w_cat, axis=0, keepdims=True)
    w_out[...] = (w_cat / (wsum + 1e-8)).T
    i_out[...] = jnp.concatenate(idxs, axis=0).T


@jax.jit
def kernel(x, W, expert_bias):
    grid = TOKENS // T_BLK
    w_t = W.T
    bias = expert_bias.reshape(NUM_EXPERTS, 1)

    w_out, i_out, psum, zsum = pl.pallas_call(
        _router_block,
        grid=(grid,),
        in_specs=[
            pl.BlockSpec((T_BLK, D_MODEL), lambda i: (i, 0)),
            pl.BlockSpec((D_MODEL, NUM_EXPERTS), lambda i: (0, 0)),
            pl.BlockSpec((NUM_EXPERTS, 1), lambda i: (0, 0)),
        ],
        out_specs=[
            pl.BlockSpec((T_BLK, TOP_K), lambda i: (i, 0)),
            pl.BlockSpec((T_BLK, TOP_K), lambda i: (i, 0)),
            pl.BlockSpec((1, 1, NUM_EXPERTS), lambda i: (i, 0, 0)),
            pl.BlockSpec((1, 1, NUM_EXPERTS), lambda i: (i, 0, 0)),
        ],
        out_shape=[
            jax.ShapeDtypeStruct((TOKENS, TOP_K), jnp.float32),
            jax.ShapeDtypeStruct((TOKENS, TOP_K), jnp.int32),
            jax.ShapeDtypeStruct((grid, 1, NUM_EXPERTS), jnp.float32),
            jax.ShapeDtypeStruct((grid, 1, NUM_EXPERTS), jnp.float32),
        ],
        compiler_params=pltpu.CompilerParams(
            dimension_semantics=("parallel",),
        ),
    )(x, w_t, bias)

    tokens_per_expert = jnp.sum(psum, axis=(0, 1)) / TOKENS
    uniform = 1.0 / NUM_EXPERTS
    load_balance_loss = (
        jnp.sum((tokens_per_expert - uniform) ** 2) * NUM_EXPERTS
    )
    router_z_loss = jnp.sum(zsum) / (TOKENS * NUM_EXPERTS) * 0.001
    return (w_out, i_out, load_balance_loss, router_z_loss)
